# baseline (speedup 1.0000x reference)
import functools

import jax
import jax.numpy as jnp
from jax import lax
from jax.experimental import pallas as pl
from jax.experimental.pallas import tpu as pltpu
from jax.experimental.pallas import tpu_sc as plsc

B = 16384
D = 32
NC = 2
NS = 16
NW = NC * NS
B_PER_W = B // NW


def _sc_gather_body(uidx_hbm, iidx_hbm, utab_hbm, itab_hbm,
                    uout_hbm, iout_hbm,
                    uidx_v, iidx_v, urows_v, irows_v, usem, isem):
    wid = lax.axis_index("s") * NC + lax.axis_index("c")
    base = wid * B_PER_W
    pltpu.sync_copy(uidx_hbm.at[pl.ds(base, B_PER_W)], uidx_v)
    pltpu.sync_copy(iidx_hbm.at[pl.ds(base, B_PER_W)], iidx_v)
    ucopy = pltpu.async_copy(utab_hbm.at[uidx_v], urows_v, usem)
    icopy = pltpu.async_copy(itab_hbm.at[iidx_v], irows_v, isem)
    ucopy.wait()
    icopy.wait()
    pltpu.sync_copy(urows_v, uout_hbm.at[pl.ds(base, B_PER_W)])
    pltpu.sync_copy(irows_v, iout_hbm.at[pl.ds(base, B_PER_W)])


_sc_gather = pl.kernel(
    _sc_gather_body,
    out_type=(
        jax.ShapeDtypeStruct((B, D), jnp.float32),
        jax.ShapeDtypeStruct((B, D), jnp.float32),
    ),
    mesh=plsc.VectorSubcoreMesh(core_axis_name="c", subcore_axis_name="s"),
    compiler_params=pltpu.CompilerParams(use_tc_tiling_on_sc=False),
    scratch_types=(
        pltpu.VMEM((B_PER_W,), jnp.int32),
        pltpu.VMEM((B_PER_W,), jnp.int32),
        pltpu.VMEM((B_PER_W, D), jnp.float32),
        pltpu.VMEM((B_PER_W, D), jnp.float32),
        pltpu.SemaphoreType.DMA,
        pltpu.SemaphoreType.DMA,
    ),
)


BLK = 2048


def _mlp_body(ue_ref, ie_ref, w1_ref, b1_ref, w2_ref, b2_ref, w3_ref, b3_ref,
              out_ref):
    ue = ue_ref[...]
    ie = ie_ref[...]
    w1 = w1_ref[...]
    x1 = (jnp.dot(ue, w1[:D, :], preferred_element_type=jnp.float32)
          + jnp.dot(ie, w1[D:, :], preferred_element_type=jnp.float32)
          + b1_ref[...])
    h1 = jnp.maximum(x1, 0.0)
    h2 = jnp.maximum(
        jnp.dot(h1, w2_ref[...], preferred_element_type=jnp.float32)
        + b2_ref[...], 0.0)
    out_ref[...] = (jnp.dot(h2, w3_ref[...], preferred_element_type=jnp.float32)
                    + b3_ref[...])


def _mlp(ue, ie, W1, b1, W2, b2, W3, b3):
    grid = (B // BLK,)
    return pl.pallas_call(
        _mlp_body,
        grid=grid,
        in_specs=[
            pl.BlockSpec((BLK, D), lambda i: (i, 0)),
            pl.BlockSpec((BLK, D), lambda i: (i, 0)),
            pl.BlockSpec((2 * D, 64), lambda i: (0, 0)),
            pl.BlockSpec((1, 64), lambda i: (0, 0)),
            pl.BlockSpec((64, 16), lambda i: (0, 0)),
            pl.BlockSpec((1, 16), lambda i: (0, 0)),
            pl.BlockSpec((16, 1), lambda i: (0, 0)),
            pl.BlockSpec((1, 1), lambda i: (0, 0)),
        ],
        out_specs=pl.BlockSpec((BLK, 1), lambda i: (i, 0)),
        out_shape=jax.ShapeDtypeStruct((B, 1), jnp.float32),
    )(ue, ie, W1, b1.reshape(1, 64), W2, b2.reshape(1, 16),
      W3, b3.reshape(1, 1))


def kernel(user_indices, item_indices, user_table, item_table,
           W1, b1, W2, b2, W3, b3):
    ue, ie = _sc_gather(user_indices, item_indices, user_table, item_table)
    out = _mlp(ue, ie, W1, b1, W2, b2, W3, b3)
    return out.reshape(B)


# speedup vs baseline: 1.4376x; 1.4376x over previous
import jax
import jax.numpy as jnp
from jax import lax
from jax.experimental import pallas as pl
from jax.experimental.pallas import tpu as pltpu
from jax.experimental.pallas import tpu_sc as plsc

B = 16384
D = 32
NC = 2
NS = 16
NW = NC * NS
B_PER_W = B // NW
G = 16
N_GROUPS = B_PER_W // G


def _sc_gather_body(uidx_hbm, iidx_hbm, utab_hbm, itab_hbm,
                    uout_hbm, iout_hbm,
                    uidx_v, iidx_v, ustage, istage, usem, isem):
    wid = lax.axis_index("s") * NC + lax.axis_index("c")
    base = wid * B_PER_W
    pltpu.sync_copy(uidx_hbm.at[pl.ds(base, B_PER_W)], uidx_v)
    pltpu.sync_copy(iidx_hbm.at[pl.ds(base, B_PER_W)], iidx_v)

    def group_body(g, _):
        uiv = uidx_v[pl.ds(g * G, G)]
        iiv = iidx_v[pl.ds(g * G, G)]
        copies = []
        for j in range(G):
            copies.append(
                pltpu.async_copy(utab_hbm.at[uiv[j]], ustage.at[j], usem))
        for j in range(G):
            copies.append(
                pltpu.async_copy(itab_hbm.at[iiv[j]], istage.at[j], isem))
        for cp in copies:
            cp.wait()
        pltpu.sync_copy(ustage, uout_hbm.at[pl.ds(base + g * G, G)])
        pltpu.sync_copy(istage, iout_hbm.at[pl.ds(base + g * G, G)])
        return 0

    lax.fori_loop(0, N_GROUPS, group_body, 0)


_sc_gather = pl.kernel(
    _sc_gather_body,
    out_type=(
        jax.ShapeDtypeStruct((B, D), jnp.float32),
        jax.ShapeDtypeStruct((B, D), jnp.float32),
    ),
    mesh=plsc.VectorSubcoreMesh(core_axis_name="c", subcore_axis_name="s"),
    compiler_params=pltpu.CompilerParams(needs_layout_passes=False),
    scratch_types=(
        pltpu.VMEM((B_PER_W,), jnp.int32),
        pltpu.VMEM((B_PER_W,), jnp.int32),
        pltpu.VMEM((G, D), jnp.float32),
        pltpu.VMEM((G, D), jnp.float32),
        pltpu.SemaphoreType.DMA,
        pltpu.SemaphoreType.DMA,
    ),
)


BLK = 2048


def _mlp_body(ue_ref, ie_ref, w1_ref, b1_ref, w2_ref, b2_ref, w3_ref, b3_ref,
              out_ref):
    ue = ue_ref[...]
    ie = ie_ref[...]
    w1 = w1_ref[...]
    x1 = (jnp.dot(ue, w1[:D, :], preferred_element_type=jnp.float32)
          + jnp.dot(ie, w1[D:, :], preferred_element_type=jnp.float32)
          + b1_ref[...])
    h1 = jnp.maximum(x1, 0.0)
    h2 = jnp.maximum(
        jnp.dot(h1, w2_ref[...], preferred_element_type=jnp.float32)
        + b2_ref[...], 0.0)
    out_ref[...] = (jnp.dot(h2, w3_ref[...], preferred_element_type=jnp.float32)
                    + b3_ref[...])


def _mlp(ue, ie, W1, b1, W2, b2, W3, b3):
    grid = (B // BLK,)
    return pl.pallas_call(
        _mlp_body,
        grid=grid,
        in_specs=[
            pl.BlockSpec((BLK, D), lambda i: (i, 0)),
            pl.BlockSpec((BLK, D), lambda i: (i, 0)),
            pl.BlockSpec((2 * D, 64), lambda i: (0, 0)),
            pl.BlockSpec((1, 64), lambda i: (0, 0)),
            pl.BlockSpec((64, 16), lambda i: (0, 0)),
            pl.BlockSpec((1, 16), lambda i: (0, 0)),
            pl.BlockSpec((16, 1), lambda i: (0, 0)),
            pl.BlockSpec((1, 1), lambda i: (0, 0)),
        ],
        out_specs=pl.BlockSpec((BLK, 1), lambda i: (i, 0)),
        out_shape=jax.ShapeDtypeStruct((B, 1), jnp.float32),
    )(ue, ie, W1, b1.reshape(1, 64), W2, b2.reshape(1, 16),
      W3, b3.reshape(1, 1))


def kernel(user_indices, item_indices, user_table, item_table,
           W1, b1, W2, b2, W3, b3):
    ue, ie = _sc_gather(user_indices, item_indices, user_table, item_table)
    out = _mlp(ue, ie, W1, b1, W2, b2, W3, b3)
    return out.reshape(B)


# speedup vs baseline: 2.9946x; 2.0830x over previous
import jax
import jax.numpy as jnp
from jax import lax
from jax.experimental import pallas as pl
from jax.experimental.pallas import tpu as pltpu
from jax.experimental.pallas import tpu_sc as plsc

B = 16384
D = 32
N = 1_000_000
CH = 16384
NFULL = N // CH
TAIL = 512
NTAIL = 64
NCH = NFULL + 1
CPS = NCH // 2
SB = 1024
NVR = SB // 16


def _sweep(idx_hbm, tabT_hbm, tail_hbm, outT_hbm, cid, sid, base,
           idx_v, spm, order_sm, cnt_sm, off_sm, flat_v, dst_v, stage_v,
           tail_v, sem, gsem):
    lanes = jnp.arange(16, dtype=jnp.int32)
    pltpu.sync_copy(tail_hbm, tail_v)

    def zbody(j, _):
        for c in range(D):
            stage_v[c, pl.ds(j * 16, 16)] = jnp.zeros((16,), jnp.float32)
        return 0
    lax.fori_loop(0, SB // 16, zbody, 0)

    pltpu.sync_copy(idx_hbm.at[pl.ds(base, SB)], idx_v)

    for b in range(NCH):
        cnt_sm[b] = 0

    def hbody(j, _):
        v = idx_v[pl.ds(j * 16, 16)]
        for l in range(16):
            b = lax.shift_right_logical(v[l], 14)
            cnt_sm[b] = cnt_sm[b] + 1
        return 0
    lax.fori_loop(0, NVR, hbody, 0)

    run = jnp.int32(0)
    for b in range(NCH):
        off_sm[b] = run
        run = run + cnt_sm[b]
    off_sm[NCH] = run
    for b in range(NCH):
        cnt_sm[b] = off_sm[b]

    def sbody(j, _):
        v = idx_v[pl.ds(j * 16, 16)]
        for l in range(16):
            val = v[l]
            b = lax.shift_right_logical(val, 14)
            o = cnt_sm[b]
            order_sm[o] = (val << 10) | (j * 16 + l)
            cnt_sm[b] = o + 1
        return 0
    lax.fori_loop(0, NVR, sbody, 0)

    def cbody(gl, _):
        g = cid * CPS + gl
        lo = g * CH
        row = 2 * sid

        @pl.when(g < NFULL)
        def _full():
            cp0 = pltpu.async_copy(tabT_hbm.at[row, pl.ds(lo, CH)],
                                   spm.at[pl.ds(row * CH, CH)], sem)
            cp1 = pltpu.async_copy(tabT_hbm.at[row + 1, pl.ds(lo, CH)],
                                   spm.at[pl.ds((row + 1) * CH, CH)], sem)
            cp0.wait()
            cp1.wait()

        @pl.when(g == NFULL)
        def _tail():
            cp0 = pltpu.async_copy(
                tabT_hbm.at[row, pl.ds(NFULL * CH, TAIL)],
                spm.at[pl.ds(row * CH, TAIL)], sem)
            cp1 = pltpu.async_copy(
                tabT_hbm.at[row + 1, pl.ds(NFULL * CH, TAIL)],
                spm.at[pl.ds((row + 1) * CH, TAIL)], sem)
            cp0.wait()
            cp1.wait()

        plsc.subcore_barrier()

        st = off_sm[g]
        n = off_sm[g + 1] - st
        limit = jnp.where(g == NFULL, TAIL, CH)

        def gbody(k, _):
            packed = jnp.zeros((16,), jnp.int32)
            for l in range(16):
                rd = jnp.minimum(st + k * 16 + l, SB - 1)
                pv = order_sm[rd]
                packed = jnp.where(lanes == l,
                                   jnp.full((16,), pv, jnp.int32), packed)
            valid = (k * 16 + lanes) < n
            bse = lax.shift_right_logical(packed, 10) - lo
            pos = lax.bitwise_and(packed, 1023)
            in_spm = valid & (bse < limit)
            for c in range(D):
                flat_v[pl.ds(c * 16, 16)] = jnp.where(
                    in_spm, bse + c * CH, -1)
            cp = pltpu.async_copy(
                spm.at[plsc.Indices(flat_v, ignored_value=-1)], dst_v, gsem)
            cp.wait()
            for c in range(D):
                vals = dst_v[pl.ds(c * 16, 16)]
                plsc.store_scatter(
                    stage_v, [jnp.full((16,), c, jnp.int32), pos],
                    vals, mask=in_spm)

            @pl.when(g == NFULL)
            def _tail_gather():
                mask_t = valid & (bse >= TAIL)
                r_off = lax.bitwise_and(bse - TAIL, NTAIL - 1)
                for c in range(D):
                    vals = plsc.load_gather(
                        tail_v, [r_off, jnp.full((16,), c, jnp.int32)])
                    plsc.store_scatter(
                        stage_v, [jnp.full((16,), c, jnp.int32), pos],
                        vals, mask=mask_t)
            return 0
        lax.fori_loop(0, (n + 15) // 16, gbody, 0)
        plsc.subcore_barrier()
        return 0

    lax.fori_loop(0, CPS, cbody, 0)
    pltpu.sync_copy(stage_v, outT_hbm.at[cid, :, pl.ds(base, SB)])


def _sc_body(uidx_hbm, iidx_hbm, utabT_hbm, itabT_hbm, utail_hbm, itail_hbm,
             uoutT_hbm, ioutT_hbm,
             idx_v, spm, order_sm, cnt_sm, off_sm, flat_v, dst_v, stage_v,
             tail_v, sem, gsem):
    cid = lax.axis_index("c")
    sid = lax.axis_index("s")
    base = sid * SB
    _sweep(uidx_hbm, utabT_hbm, utail_hbm, uoutT_hbm, cid, sid, base,
           idx_v, spm, order_sm, cnt_sm, off_sm, flat_v, dst_v, stage_v,
           tail_v, sem, gsem)
    _sweep(iidx_hbm, itabT_hbm, itail_hbm, ioutT_hbm, cid, sid, base,
           idx_v, spm, order_sm, cnt_sm, off_sm, flat_v, dst_v, stage_v,
           tail_v, sem, gsem)


_sc_gather = pl.kernel(
    _sc_body,
    out_type=(
        jax.ShapeDtypeStruct((2, D, B), jnp.float32),
        jax.ShapeDtypeStruct((2, D, B), jnp.float32),
    ),
    mesh=plsc.VectorSubcoreMesh(core_axis_name="c", subcore_axis_name="s"),
    compiler_params=pltpu.CompilerParams(needs_layout_passes=False),
    scratch_types=(
        pltpu.VMEM((SB,), jnp.int32),
        pltpu.VMEM_SHARED((D * CH,), jnp.float32),
        pltpu.SMEM((SB,), jnp.int32),
        pltpu.SMEM((NCH + 2,), jnp.int32),
        pltpu.SMEM((NCH + 2,), jnp.int32),
        pltpu.VMEM((16 * D,), jnp.int32),
        pltpu.VMEM((16 * D,), jnp.float32),
        pltpu.VMEM((D, SB), jnp.float32),
        pltpu.VMEM((NTAIL, D), jnp.float32),
        pltpu.SemaphoreType.DMA,
        pltpu.SemaphoreType.DMA,
    ),
)


BLK = 2048


def _mlp_body(ueT_ref, ieT_ref, w1_ref, b1_ref, w2_ref, b2_ref, w3_ref,
              b3_ref, out_ref):
    u2 = ueT_ref[...]
    i2 = ieT_ref[...]
    xu = u2[0] + u2[1]
    xi = i2[0] + i2[1]
    w1 = w1_ref[...]
    cdims = (((0,), (0,)), ((), ()))
    x1 = (lax.dot_general(w1[:D, :], xu, cdims,
                          preferred_element_type=jnp.float32)
          + lax.dot_general(w1[D:, :], xi, cdims,
                            preferred_element_type=jnp.float32)
          + b1_ref[...])
    h1 = jnp.maximum(x1, 0.0)
    h2 = jnp.maximum(
        lax.dot_general(w2_ref[...], h1, cdims,
                        preferred_element_type=jnp.float32)
        + b2_ref[...], 0.0)
    out_ref[...] = (lax.dot_general(w3_ref[...], h2, cdims,
                                    preferred_element_type=jnp.float32)
                    + b3_ref[...])


def _mlp(ueT2, ieT2, W1, b1, W2, b2, W3, b3):
    grid = (B // BLK,)
    return pl.pallas_call(
        _mlp_body,
        grid=grid,
        in_specs=[
            pl.BlockSpec((2, D, BLK), lambda i: (0, 0, i)),
            pl.BlockSpec((2, D, BLK), lambda i: (0, 0, i)),
            pl.BlockSpec((2 * D, 64), lambda i: (0, 0)),
            pl.BlockSpec((64, 1), lambda i: (0, 0)),
            pl.BlockSpec((64, 16), lambda i: (0, 0)),
            pl.BlockSpec((16, 1), lambda i: (0, 0)),
            pl.BlockSpec((16, 1), lambda i: (0, 0)),
            pl.BlockSpec((1, 1), lambda i: (0, 0)),
        ],
        out_specs=pl.BlockSpec((1, BLK), lambda i: (0, i)),
        out_shape=jax.ShapeDtypeStruct((1, B), jnp.float32),
    )(ueT2, ieT2, W1, b1.reshape(64, 1), W2, b2.reshape(16, 1),
      W3, b3.reshape(1, 1))


def kernel(user_indices, item_indices, user_table, item_table,
           W1, b1, W2, b2, W3, b3):
    utail = lax.slice(user_table, (N - NTAIL, 0), (N, D))
    itail = lax.slice(item_table, (N - NTAIL, 0), (N, D))
    ueT2, ieT2 = _sc_gather(user_indices, item_indices,
                            user_table.T, item_table.T, utail, itail)
    out = _mlp(ueT2, ieT2, W1, b1, W2, b2, W3, b3)
    return out.reshape(B)


# speedup vs baseline: 3.8327x; 1.2799x over previous
import jax
import jax.numpy as jnp
from jax import lax
from jax.experimental import pallas as pl
from jax.experimental.pallas import tpu as pltpu
from jax.experimental.pallas import tpu_sc as plsc

B = 16384
D = 32
N = 1_000_000
CH = 16384
NFULL = N // CH
TAIL = 512
NTAIL = 64
NCH = NFULL + 1
CPS = NCH // 2
SB = 1024
NVR = SB // 16


def _sweep(idx_hbm, tabT_hbm, tail_hbm, outT_hbm, cid, sid, base,
           idx_v, spm, order_sm, cnt_sm, off_sm, flat_v, dst_v, stage_v,
           tail_v, sem, gsem, gsem2):
    lanes = jnp.arange(16, dtype=jnp.int32)
    pltpu.sync_copy(tail_hbm, tail_v)

    def zbody(j, _):
        for c in range(D):
            stage_v[c, pl.ds(j * 16, 16)] = jnp.zeros((16,), jnp.float32)
        return 0
    lax.fori_loop(0, SB // 16, zbody, 0)

    pltpu.sync_copy(idx_hbm.at[pl.ds(base, SB)], idx_v)

    for b in range(NCH):
        cnt_sm[b] = 0

    def hbody(j, _):
        v = idx_v[pl.ds(j * 16, 16)]
        for l in range(16):
            b = lax.shift_right_logical(v[l], 14)
            cnt_sm[b] = cnt_sm[b] + 1
        return 0
    lax.fori_loop(0, NVR, hbody, 0)

    run = jnp.int32(0)
    for b in range(NCH):
        off_sm[b] = run
        run = run + cnt_sm[b]
    off_sm[NCH] = run
    for b in range(NCH):
        cnt_sm[b] = off_sm[b]

    def sbody(j, _):
        v = idx_v[pl.ds(j * 16, 16)]
        for l in range(16):
            val = v[l]
            b = lax.shift_right_logical(val, 14)
            o = cnt_sm[b]
            order_sm[o] = (val << 10) | (j * 16 + l)
            cnt_sm[b] = o + 1
        return 0
    lax.fori_loop(0, NVR, sbody, 0)

    row = 2 * sid

    def _fire(g, boff, s):
        @pl.when(g < NFULL)
        def _full():
            pltpu.async_copy(tabT_hbm.at[row, pl.ds(g * CH, CH)],
                             spm.at[pl.ds(boff + row * CH, CH)], s)
            pltpu.async_copy(tabT_hbm.at[row + 1, pl.ds(g * CH, CH)],
                             spm.at[pl.ds(boff + (row + 1) * CH, CH)], s)

        @pl.when(g == NFULL)
        def _tl():
            pltpu.async_copy(tabT_hbm.at[row, pl.ds(NFULL * CH, TAIL)],
                             spm.at[pl.ds(boff + row * CH, TAIL)], s)
            pltpu.async_copy(tabT_hbm.at[row + 1, pl.ds(NFULL * CH, TAIL)],
                             spm.at[pl.ds(boff + (row + 1) * CH, TAIL)], s)

    def _drain(g, boff, s):
        @pl.when(g < NFULL)
        def _full():
            pltpu.make_async_copy(tabT_hbm.at[row, pl.ds(g * CH, CH)],
                                  spm.at[pl.ds(boff + row * CH, CH)],
                                  s).wait()
            pltpu.make_async_copy(tabT_hbm.at[row + 1, pl.ds(g * CH, CH)],
                                  spm.at[pl.ds(boff + (row + 1) * CH, CH)],
                                  s).wait()

        @pl.when(g == NFULL)
        def _tl():
            pltpu.make_async_copy(tabT_hbm.at[row, pl.ds(NFULL * CH, TAIL)],
                                  spm.at[pl.ds(boff + row * CH, TAIL)],
                                  s).wait()
            pltpu.make_async_copy(
                tabT_hbm.at[row + 1, pl.ds(NFULL * CH, TAIL)],
                spm.at[pl.ds(boff + (row + 1) * CH, TAIL)], s).wait()

    _fire(cid * CPS, 0, sem)

    def cbody(gl, _):
        g = cid * CPS + gl
        lo = g * CH
        par = lax.bitwise_and(gl, 1)
        poff = par * (D * CH)
        noff = (1 - par) * (D * CH)

        @pl.when(gl + 1 < CPS)
        def _prefetch():
            @pl.when(par == 0)
            def _pe():
                _fire(g + 1, noff, gsem2)

            @pl.when(par == 1)
            def _po():
                _fire(g + 1, noff, sem)

        @pl.when(par == 0)
        def _de():
            _drain(g, poff, sem)

        @pl.when(par == 1)
        def _do():
            _drain(g, poff, gsem2)

        plsc.subcore_barrier()

        st = off_sm[g]
        n = off_sm[g + 1] - st
        limit = jnp.where(g == NFULL, TAIL, CH)

        def gbody(k, _):
            packed = jnp.zeros((16,), jnp.int32)
            for l in range(16):
                rd = jnp.minimum(st + k * 16 + l, SB - 1)
                pv = order_sm[rd]
                packed = jnp.where(lanes == l,
                                   jnp.full((16,), pv, jnp.int32), packed)
            valid = (k * 16 + lanes) < n
            bse = lax.shift_right_logical(packed, 10) - lo
            pos = lax.bitwise_and(packed, 1023)
            in_spm = valid & (bse < limit)
            for c in range(D):
                flat_v[pl.ds(c * 16, 16)] = jnp.where(
                    in_spm, poff + bse + c * CH, -1)
            cp = pltpu.async_copy(
                spm.at[plsc.Indices(flat_v, ignored_value=-1)], dst_v, gsem)
            cp.wait()
            for c in range(D):
                vals = dst_v[pl.ds(c * 16, 16)]
                plsc.store_scatter(
                    stage_v, [jnp.full((16,), c, jnp.int32), pos],
                    vals, mask=in_spm)

            @pl.when(g == NFULL)
            def _tail_gather():
                mask_t = valid & (bse >= TAIL)
                r_off = lax.bitwise_and(bse - TAIL, NTAIL - 1)
                for c in range(D):
                    vals = plsc.load_gather(
                        tail_v, [r_off, jnp.full((16,), c, jnp.int32)])
                    plsc.store_scatter(
                        stage_v, [jnp.full((16,), c, jnp.int32), pos],
                        vals, mask=mask_t)
            return 0
        lax.fori_loop(0, (n + 15) // 16, gbody, 0)
        plsc.subcore_barrier()
        return 0

    lax.fori_loop(0, CPS, cbody, 0)
    pltpu.sync_copy(stage_v, outT_hbm.at[cid, :, pl.ds(base, SB)])


def _sc_body(uidx_hbm, iidx_hbm, utabT_hbm, itabT_hbm, utail_hbm, itail_hbm,
             uoutT_hbm, ioutT_hbm,
             idx_v, spm, order_sm, cnt_sm, off_sm, flat_v, dst_v, stage_v,
             tail_v, sem, gsem, gsem2):
    cid = lax.axis_index("c")
    sid = lax.axis_index("s")
    base = sid * SB
    _sweep(uidx_hbm, utabT_hbm, utail_hbm, uoutT_hbm, cid, sid, base,
           idx_v, spm, order_sm, cnt_sm, off_sm, flat_v, dst_v, stage_v,
           tail_v, sem, gsem, gsem2)
    _sweep(iidx_hbm, itabT_hbm, itail_hbm, ioutT_hbm, cid, sid, base,
           idx_v, spm, order_sm, cnt_sm, off_sm, flat_v, dst_v, stage_v,
           tail_v, sem, gsem, gsem2)


_sc_gather = pl.kernel(
    _sc_body,
    out_type=(
        jax.ShapeDtypeStruct((2, D, B), jnp.float32),
        jax.ShapeDtypeStruct((2, D, B), jnp.float32),
    ),
    mesh=plsc.VectorSubcoreMesh(core_axis_name="c", subcore_axis_name="s"),
    compiler_params=pltpu.CompilerParams(needs_layout_passes=False),
    scratch_types=(
        pltpu.VMEM((SB,), jnp.int32),
        pltpu.VMEM_SHARED((2 * D * CH,), jnp.float32),
        pltpu.SMEM((SB,), jnp.int32),
        pltpu.SMEM((NCH + 2,), jnp.int32),
        pltpu.SMEM((NCH + 2,), jnp.int32),
        pltpu.VMEM((16 * D,), jnp.int32),
        pltpu.VMEM((16 * D,), jnp.float32),
        pltpu.VMEM((D, SB), jnp.float32),
        pltpu.VMEM((NTAIL, D), jnp.float32),
        pltpu.SemaphoreType.DMA,
        pltpu.SemaphoreType.DMA,
        pltpu.SemaphoreType.DMA,
    ),
)


BLK = 2048


def _mlp_body(ueT_ref, ieT_ref, w1_ref, b1_ref, w2_ref, b2_ref, w3_ref,
              b3_ref, out_ref):
    u2 = ueT_ref[...]
    i2 = ieT_ref[...]
    xu = u2[0] + u2[1]
    xi = i2[0] + i2[1]
    w1 = w1_ref[...]
    cdims = (((0,), (0,)), ((), ()))
    x1 = (lax.dot_general(w1[:D, :], xu, cdims,
                          preferred_element_type=jnp.float32)
          + lax.dot_general(w1[D:, :], xi, cdims,
                            preferred_element_type=jnp.float32)
          + b1_ref[...])
    h1 = jnp.maximum(x1, 0.0)
    h2 = jnp.maximum(
        lax.dot_general(w2_ref[...], h1, cdims,
                        preferred_element_type=jnp.float32)
        + b2_ref[...], 0.0)
    out_ref[...] = (lax.dot_general(w3_ref[...], h2, cdims,
                                    preferred_element_type=jnp.float32)
                    + b3_ref[...])


def _mlp(ueT2, ieT2, W1, b1, W2, b2, W3, b3):
    grid = (B // BLK,)
    return pl.pallas_call(
        _mlp_body,
        grid=grid,
        in_specs=[
            pl.BlockSpec((2, D, BLK), lambda i: (0, 0, i)),
            pl.BlockSpec((2, D, BLK), lambda i: (0, 0, i)),
            pl.BlockSpec((2 * D, 64), lambda i: (0, 0)),
            pl.BlockSpec((64, 1), lambda i: (0, 0)),
            pl.BlockSpec((64, 16), lambda i: (0, 0)),
            pl.BlockSpec((16, 1), lambda i: (0, 0)),
            pl.BlockSpec((16, 1), lambda i: (0, 0)),
            pl.BlockSpec((1, 1), lambda i: (0, 0)),
        ],
        out_specs=pl.BlockSpec((1, BLK), lambda i: (0, i)),
        out_shape=jax.ShapeDtypeStruct((1, B), jnp.float32),
    )(ueT2, ieT2, W1, b1.reshape(64, 1), W2, b2.reshape(16, 1),
      W3, b3.reshape(1, 1))


def kernel(user_indices, item_indices, user_table, item_table,
           W1, b1, W2, b2, W3, b3):
    utail = lax.slice(user_table, (N - NTAIL, 0), (N, D))
    itail = lax.slice(item_table, (N - NTAIL, 0), (N, D))
    ueT2, ieT2 = _sc_gather(user_indices, item_indices,
                            user_table.T, item_table.T, utail, itail)
    out = _mlp(ueT2, ieT2, W1, b1, W2, b2, W3, b3)
    return out.reshape(B)
